# baseline (speedup 1.0000x reference)
import functools
from math import ceil

import jax
import jax.numpy as jnp
from jax import lax
from jax.experimental import pallas as pl
from jax.experimental.pallas import tpu as pltpu, tpu_sc as plsc

ALPHA = 0.9
K_SEL = 40
N_ROWS = 128
V = 100000
K_BIG = int(ceil((1.0 - ALPHA) * V))

NC, NS, L = 2, 16, 16
NW = NC * NS
ROWS_PER_W = N_ROWS // NW

NVEC = V // L
HBITS = 11
NBINS1 = 1 << HBITS
SHIFT1 = 32 - HBITS
NBINS2 = 2048
NBINS3 = 1024
CAPK = 12288
CAP40 = 512

_MIN32 = -0x80000000
_M31 = 0x7FFFFFFF


def _ordmap_f(v):
    b = lax.bitcast_convert_type(v, jnp.int32)
    return b ^ ((b >> 31) & _M31)


def _unmap_f(s):
    i = s ^ ((s >> 31) & _M31)
    return lax.bitcast_convert_type(i, jnp.float32)


def _ub(s):
    return s ^ _MIN32


def _poly_ln(x):
    b = lax.bitcast_convert_type(x, jnp.int32)
    e = ((b >> 23) & jnp.int32(0xFF)) - jnp.int32(127)
    m = lax.bitcast_convert_type((b & jnp.int32(0x7FFFFF)) | jnp.int32(0x3F800000),
                                 jnp.float32)
    big = m > jnp.float32(1.4142135381698608)
    m = jnp.where(big, m * jnp.float32(0.5), m)
    e = e + big.astype(jnp.int32)
    s = (m - jnp.float32(1.0)) / (m + jnp.float32(1.0))
    s2 = s * s
    t = jnp.float32(1.0 / 7.0) + s2 * jnp.float32(1.0 / 9.0)
    t = jnp.float32(1.0 / 5.0) + s2 * t
    t = jnp.float32(1.0 / 3.0) + s2 * t
    t = jnp.float32(1.0) + s2 * t
    t = jnp.float32(2.0) * s * t
    return e.astype(jnp.float32) * jnp.float32(0.6931471805599453) + t


def _zero_i32(ref, n):
    z = jnp.zeros((L,), jnp.int32)

    def body(i, c):
        ref[pl.ds(i * L, L)] = z
        return c

    lax.fori_loop(0, n // L, body, 0)


def _scan_hist(hist_ref, nbins, k):
    iota = lax.iota(jnp.int32, L)

    def body(i, carry):
        B, gt, acc, found = carry
        j = nbins // L - 1 - i
        h = hist_ref[pl.ds(j * L, L)]
        hr = lax.rev(h, (0,))
        cs = plsc.cumsum(hr)
        tot = acc + cs
        hit = tot >= k
        npc = plsc.all_reduce_population_count(hit)
        ffs = plsc.all_reduce_ffs(hit)
        sel = iota == ffs
        cnt_at = jnp.sum(jnp.where(sel, tot, 0))
        h_at = jnp.sum(jnp.where(sel, hr, 0))
        binc = j * L + 15 - ffs
        bin_s = jnp.sum(jnp.where(sel, binc, 0))
        upd = (jnp.sum(jnp.where(iota == 0, npc, 0)) > 0) & (found == 0)
        B = jnp.where(upd, bin_s, B)
        gt = jnp.where(upd, cnt_at - h_at, gt)
        found = jnp.where(upd, 1, found)
        acc = acc + jnp.sum(h)
        return B, gt, acc, found

    B, gt, _, _ = lax.fori_loop(
        0, nbins // L, body,
        (jnp.int32(0), jnp.int32(0), jnp.int32(0), jnp.int32(0)))
    return B, gt


def _count_ge_arr(src_ref, n, thr_s):
    iota = lax.iota(jnp.int32, L)
    nv = (n + L - 1) // L

    def body(i, cnt):
        s = src_ref[pl.ds(i * L, L)]
        valid = (i * L + iota) < n
        m = valid & (s >= thr_s)
        return cnt + plsc.all_reduce_population_count(m)

    cnt = lax.fori_loop(0, nv, body, jnp.zeros((L,), jnp.int32))
    return jnp.sum(jnp.where(iota == 0, cnt, 0))


def _kth_largest_arr(src_ref, n, k):
    cnt0 = _count_ge_arr(src_ref, n, jnp.int32(0))
    T = jnp.where(cnt0 >= k, jnp.int32(0), _MIN32)

    def body(it, T):
        cand = T | (jnp.int32(1) << (30 - it))
        cnt = _count_ge_arr(src_ref, n, cand)
        return jnp.where(cnt >= k, cand, T)

    return lax.fori_loop(0, 31, body, T)


def _process_tensor(row_v, hist_v, candk_v, collect40, c40v_v, c40i_v):
    iota = lax.iota(jnp.int32, L)
    ones = jnp.ones((L,), jnp.int32)

    _zero_i32(hist_v, NBINS1)

    def p1(i, c):
        v = row_v[pl.ds(i * L, L)]
        s = _ordmap_f(v)
        bins = lax.shift_right_logical(_ub(s), SHIFT1)
        plsc.addupdate_scatter(hist_v, [bins], ones)
        return c

    lax.fori_loop(0, NVEC, p1, 0)

    B1, gt1 = _scan_hist(hist_v, NBINS1, jnp.int32(K_BIG))
    if collect40:
        B140, gt140 = _scan_hist(hist_v, NBINS1, jnp.int32(K_SEL))
    tlo = _unmap_f(lax.shift_left(B1, SHIFT1) ^ _MIN32)

    def p2(i, carry):
        bk, b40, acc = carry
        v = row_v[pl.ds(i * L, L)]
        s = _ordmap_f(v)
        bins = lax.shift_right_logical(_ub(s), SHIFT1)
        e = jnp.exp(v - tlo)
        acc = acc + jnp.where(bins > B1, e, jnp.float32(0.0))
        mk = bins == B1
        offk = bk + plsc.cumsum(mk.astype(jnp.int32)) - 1
        plsc.store_scatter(candk_v, [offk], s, mask=mk & (offk < CAPK))
        bk = bk + plsc.all_reduce_population_count(mk)
        if collect40:
            m40 = bins >= B140
            off4 = b40 + plsc.cumsum(m40.astype(jnp.int32)) - 1
            ok4 = m40 & (off4 < CAP40)
            plsc.store_scatter(c40v_v, [off4], s, mask=ok4)
            plsc.store_scatter(c40i_v, [off4], i * L + iota, mask=ok4)
            b40 = b40 + plsc.all_reduce_population_count(m40)
        return bk, b40, acc

    z0 = jnp.zeros((L,), jnp.int32)
    bk, b40, acc = lax.fori_loop(0, NVEC, p2,
                                 (z0, z0, jnp.zeros((L,), jnp.float32)))
    nk = jnp.minimum(jnp.sum(jnp.where(iota == 0, bk, 0)), CAPK)
    acc_hi = jnp.sum(acc)

    _zero_i32(hist_v, NBINS2)
    nkv = (nk + L - 1) // L

    def m2(i, c):
        s = candk_v[pl.ds(i * L, L)]
        valid = (i * L + iota) < nk
        bins = lax.shift_right_logical(_ub(s), 10) & jnp.int32(NBINS2 - 1)
        plsc.addupdate_scatter(hist_v, [bins], ones, mask=valid)
        return c

    lax.fori_loop(0, nkv, m2, 0)
    k2 = jnp.int32(K_BIG) - gt1
    B2, gt2 = _scan_hist(hist_v, NBINS2, k2)

    _zero_i32(hist_v, NBINS3)

    def m3(i, c):
        s = candk_v[pl.ds(i * L, L)]
        valid = (i * L + iota) < nk
        b2 = lax.shift_right_logical(_ub(s), 10) & jnp.int32(NBINS2 - 1)
        bins = _ub(s) & jnp.int32(NBINS3 - 1)
        plsc.addupdate_scatter(hist_v, [bins], ones, mask=valid & (b2 == B2))
        return c

    lax.fori_loop(0, nkv, m3, 0)
    k3 = k2 - gt2
    B3, _ = _scan_hist(hist_v, NBINS3, k3)

    t_ub = lax.shift_left(B1, SHIFT1) | lax.shift_left(B2, 10) | B3
    t_s = t_ub ^ _MIN32
    t_f = _unmap_f(t_s)

    def tail(i, carry):
        cge, zt = carry
        s = candk_v[pl.ds(i * L, L)]
        valid = (i * L + iota) < nk
        m = valid & (s >= t_s)
        cge = cge + plsc.all_reduce_population_count(m)
        v = _unmap_f(s)
        zt = zt + jnp.where(m, jnp.exp(v - tlo), jnp.float32(0.0))
        return cge, zt

    cge_t, zt = lax.fori_loop(0, nkv, tail, (z0, jnp.zeros((L,), jnp.float32)))
    cge = gt1 + jnp.sum(jnp.where(iota == 0, cge_t, 0))
    corr = jnp.sum(jnp.where(iota == 0,
                             jnp.exp(jnp.full((L,), t_f - tlo, jnp.float32)),
                             jnp.float32(0.0)))
    z = acc_hi + jnp.sum(zt) \
        - (cge - jnp.int32(K_BIG)).astype(jnp.float32) * corr

    if collect40:
        n40 = jnp.minimum(jnp.sum(jnp.where(iota == 0, b40, 0)), CAP40)
        return t_s, tlo, z, n40
    return t_s, tlo, z, jnp.int32(0)


def _sc_body(exp_hbm, ama_hbm, out_hbm, row_v, hist_v, candk_v,
             c40v_v, c40i_v, sel_s_v, sel_i_v, score_v):
    iota = lax.iota(jnp.int32, L)
    wid = lax.axis_index("s") * NC + lax.axis_index("c")

    def row_body(r, carry):
        row = wid * ROWS_PER_W + r

        pltpu.sync_copy(exp_hbm.at[row], row_v)
        te_s, tlo_e, z_e, n40 = _process_tensor(
            row_v, hist_v, candk_v, True, c40v_v, c40i_v)

        T40 = _kth_largest_arr(c40v_v, n40, jnp.int32(K_SEL))
        cnt_gt = _count_ge_arr(c40v_v, n40, T40 + 1)
        need = jnp.int32(K_SEL) - cnt_gt
        nv40 = (n40 + L - 1) // L

        sel_s_v[pl.ds(0, L)] = jnp.zeros((L,), jnp.int32)
        sel_s_v[pl.ds(L, L)] = jnp.zeros((L,), jnp.int32)
        sel_s_v[pl.ds(2 * L, L)] = jnp.zeros((L,), jnp.int32)
        sel_i_v[pl.ds(0, L)] = jnp.zeros((L,), jnp.int32)
        sel_i_v[pl.ds(L, L)] = jnp.zeros((L,), jnp.int32)
        sel_i_v[pl.ds(2 * L, L)] = jnp.zeros((L,), jnp.int32)

        def selbody(i, carry):
            kb, tb = carry
            s = c40v_v[pl.ds(i * L, L)]
            ix = c40i_v[pl.ds(i * L, L)]
            valid = (i * L + iota) < n40
            mgt = valid & (s > T40)
            mtie = valid & (s == T40)
            trank = tb + plsc.cumsum(mtie.astype(jnp.int32)) - 1
            keep = mgt | (mtie & (trank < need))
            offs = kb + plsc.cumsum(keep.astype(jnp.int32)) - 1
            okm = keep & (offs < jnp.int32(3 * L))
            plsc.store_scatter(sel_s_v, [offs], s, mask=okm)
            plsc.store_scatter(sel_i_v, [offs], ix, mask=okm)
            kb = kb + plsc.all_reduce_population_count(keep)
            tb = tb + plsc.all_reduce_population_count(mtie)
            return kb, tb

        z0 = jnp.zeros((L,), jnp.int32)
        lax.fori_loop(0, nv40, selbody, (z0, z0))

        pltpu.sync_copy(ama_hbm.at[row], row_v)
        ta_s, tlo_a, z_a, _ = _process_tensor(
            row_v, hist_v, candk_v, False, c40v_v, c40i_v)

        for j in range(3):
            s40 = sel_s_v[pl.ds(j * L, L)]
            i40 = sel_i_v[pl.ds(j * L, L)]
            v40 = _unmap_f(s40)
            pe = jnp.exp(v40 - tlo_e) / z_e
            la = plsc.load_gather(row_v, [i40])
            sa = _ordmap_f(la)
            pa = jnp.where(sa >= ta_s, jnp.exp(la - tlo_a),
                           jnp.float32(0.0)) / z_a
            ratio = pe / (pa + jnp.float32(1e-8))
            score_v[pl.ds(j * L, L)] = _poly_ln(ratio)

        ninf = jnp.full((L,), -jnp.inf, jnp.float32)

        def memset(i, c):
            row_v[pl.ds(i * L, L)] = ninf
            return c

        lax.fori_loop(0, NVEC, memset, 0)
        for j in range(3):
            i40 = sel_i_v[pl.ds(j * L, L)]
            sc = score_v[pl.ds(j * L, L)]
            slot = j * L + iota
            plsc.store_scatter(row_v, [i40], sc, mask=slot < jnp.int32(K_SEL))
        pltpu.sync_copy(row_v, out_hbm.at[row])
        return carry

    lax.fori_loop(0, ROWS_PER_W, row_body, 0)


def _make_sc_kernel():
    mesh = plsc.VectorSubcoreMesh(core_axis_name="c", subcore_axis_name="s")
    return pl.kernel(
        _sc_body,
        out_type=[jax.ShapeDtypeStruct((N_ROWS, V), jnp.float32)],
        mesh=mesh,
        scratch_types=[
            pltpu.VMEM((V,), jnp.float32),
            pltpu.VMEM((NBINS1,), jnp.int32),
            pltpu.VMEM((CAPK,), jnp.int32),
            pltpu.VMEM((CAP40,), jnp.int32),
            pltpu.VMEM((CAP40,), jnp.int32),
            pltpu.VMEM((3 * L,), jnp.int32),
            pltpu.VMEM((3 * L,), jnp.int32),
            pltpu.VMEM((3 * L,), jnp.float32),
        ],
        compiler_params=pltpu.CompilerParams(needs_layout_passes=False),
    )


def kernel(logits_exp, logits_ama):
    (out,) = _make_sc_kernel()(logits_exp, logits_ama)
    return out
